# baseline (speedup 1.0000x reference)
import jax
import jax.numpy as jnp
from jax import lax
from jax.experimental import pallas as pl
from jax.experimental.pallas import tpu as pltpu

N_EXP = 8
D = 768
N_TOK = 32768
CHUNK = 2048
NCH = N_TOK // CHUNK
NBUF = 4


def _route_chunk(w, x):
    lg = lax.dot_general(
        w, x,
        dimension_numbers=(((1,), (1,)), ((), ())),
        preferred_element_type=jnp.float32,
    )
    m = jnp.max(lg, axis=0, keepdims=True)
    ex = jnp.exp(lg - m)
    tot = jnp.sum(ex, axis=0, keepdims=True)
    rows = [ex[e:e + 1] for e in range(N_EXP)]
    v1 = rows[0]
    i1 = jnp.zeros((1, CHUNK), jnp.int32)
    for e in range(1, N_EXP):
        gt = rows[e] > v1
        v1 = jnp.where(gt, rows[e], v1)
        i1 = jnp.where(gt, jnp.int32(e), i1)
    v2 = jnp.full((1, CHUNK), -1.0, jnp.float32)
    i2 = jnp.zeros((1, CHUNK), jnp.int32)
    for e in range(N_EXP):
        ok = (rows[e] > v2) & (i1 != jnp.int32(e))
        v2 = jnp.where(ok, rows[e], v2)
        i2 = jnp.where(ok, jnp.int32(e), i2)
    s = jnp.concatenate([v1, v2], axis=0) / tot
    si = jnp.concatenate([i1, i2], axis=0)
    return s, si


def _copy_in(x_hbm, x_scr, sems, c, b):
    return pltpu.make_async_copy(
        x_hbm.at[pl.ds(c * CHUNK, CHUNK), :], x_scr.at[b], sems.at[b])


def _body(w_ref, x_hbm, os_ref, oi_ref, x_scr, sems):
    w = w_ref[...]
    for b in range(NBUF):
        _copy_in(x_hbm, x_scr, sems, b, b).start()

    for c in range(NCH):
        b = c % NBUF
        _copy_in(x_hbm, x_scr, sems, c, b).wait()
        s, si = _route_chunk(w, x_scr[b])
        os_ref[:, pl.ds(c * CHUNK, CHUNK)] = s
        oi_ref[:, pl.ds(c * CHUNK, CHUNK)] = si
        nxt = c + NBUF
        if nxt < NCH:
            _copy_in(x_hbm, x_scr, sems, nxt, b).start()


def kernel(tokens, W):
    s, si = pl.pallas_call(
        _body,
        in_specs=[
            pl.BlockSpec(memory_space=pltpu.MemorySpace.VMEM),
            pl.BlockSpec(memory_space=pl.ANY),
        ],
        out_specs=[
            pl.BlockSpec(memory_space=pltpu.MemorySpace.VMEM),
            pl.BlockSpec(memory_space=pltpu.MemorySpace.VMEM),
        ],
        out_shape=[
            jax.ShapeDtypeStruct((2, N_TOK), jnp.float32),
            jax.ShapeDtypeStruct((2, N_TOK), jnp.int32),
        ],
        scratch_shapes=[
            pltpu.VMEM((NBUF, CHUNK, D), jnp.float32),
            pltpu.SemaphoreType.DMA((NBUF,)),
        ],
    )(W, tokens)
    return s.T, si.T


# speedup vs baseline: 1.0312x; 1.0312x over previous
import jax
import jax.numpy as jnp
from jax import lax
from jax.experimental import pallas as pl

N_EXP = 8
D = 768
N_TOK = 32768
BLK = 4096
GRID = N_TOK // BLK


def _body(w_ref, x_ref, os_ref, oi_ref):
    lg = lax.dot_general(
        w_ref[...], x_ref[...],
        dimension_numbers=(((1,), (1,)), ((), ())),
        preferred_element_type=jnp.float32,
    )
    m = jnp.max(lg, axis=0, keepdims=True)
    ex = jnp.exp(lg - m)
    tot = jnp.sum(ex, axis=0, keepdims=True)
    rows = [ex[e:e + 1] for e in range(N_EXP)]
    v1 = rows[0]
    i1 = jnp.zeros((1, BLK), jnp.int32)
    for e in range(1, N_EXP):
        gt = rows[e] > v1
        v1 = jnp.where(gt, rows[e], v1)
        i1 = jnp.where(gt, jnp.int32(e), i1)
    v2 = jnp.full((1, BLK), -1.0, jnp.float32)
    i2 = jnp.zeros((1, BLK), jnp.int32)
    for e in range(N_EXP):
        ok = (rows[e] > v2) & (i1 != jnp.int32(e))
        v2 = jnp.where(ok, rows[e], v2)
        i2 = jnp.where(ok, jnp.int32(e), i2)
    s = jnp.concatenate([v1, v2], axis=0) / tot
    si = jnp.concatenate([i1, i2], axis=0)
    os_ref[...] = s
    oi_ref[...] = si


def kernel(tokens, W):
    s, si = pl.pallas_call(
        _body,
        grid=(GRID,),
        in_specs=[
            pl.BlockSpec((N_EXP, D), lambda i: (0, 0)),
            pl.BlockSpec((BLK, D), lambda i: (i, 0)),
        ],
        out_specs=[
            pl.BlockSpec((2, BLK), lambda i: (0, i)),
            pl.BlockSpec((2, BLK), lambda i: (0, i)),
        ],
        out_shape=[
            jax.ShapeDtypeStruct((2, N_TOK), jnp.float32),
            jax.ShapeDtypeStruct((2, N_TOK), jnp.int32),
        ],
    )(W, tokens)
    return s.T, si.T
